# baseline (speedup 1.0000x reference)
import functools

import jax
import jax.numpy as jnp
import numpy as np
from jax import lax
from jax.experimental import pallas as pl
from jax.experimental.pallas import tpu as pltpu
from jax.experimental.pallas import tpu_sc as plsc

V = 1000000
D = 64
BSZ = 512
NBK = (V + BSZ - 1) // BSZ
NBKP = 2048
NW = 32
LANES = 16


def _pos_encoding_np(seq_len: int, d_model: int) -> np.ndarray:
    pos = np.arange(seq_len)[:, np.newaxis]
    i = np.arange(d_model)[np.newaxis, :]
    angle_rates = 1 / np.power(10000, 2 * (i // 2) / np.float32(d_model))
    angle_rads = pos * angle_rates
    angle_rads[:, 0::2] = np.sin(angle_rads[:, 0::2])
    angle_rads[:, 1::2] = np.cos(angle_rads[:, 1::2])
    return angle_rads.astype(np.float32)


def _count(idx_flat, n):
    per_w = n // NW
    iters = per_w // LANES
    mesh = plsc.VectorSubcoreMesh(core_axis_name="c", subcore_axis_name="s")

    @functools.partial(
        pl.kernel,
        mesh=mesh,
        out_type=(jax.ShapeDtypeStruct((NBKP * NW,), jnp.int32),
                  jax.ShapeDtypeStruct((n,), jnp.int32),
                  jax.ShapeDtypeStruct((n,), jnp.int32)),
        scratch_types=[
            pltpu.VMEM((per_w,), jnp.int32),
            pltpu.VMEM((NBKP,), jnp.int32),
            pltpu.VMEM((per_w,), jnp.int32),
            pltpu.VMEM((per_w,), jnp.int32),
            pltpu.SemaphoreType.DMA,
        ],
        compiler_params=pltpu.CompilerParams(needs_layout_passes=False),
    )
    def k1a(idx_hbm, counts_hbm, pk_hbm, pv_hbm,
            idx_v, cnt_v, pk_v, pv_v, sem):
        w = lax.axis_index("c") * 16 + lax.axis_index("s")
        base_k = w * per_w
        pltpu.sync_copy(idx_hbm.at[pl.ds(base_k, per_w)], idx_v)

        zero = jnp.zeros((LANES,), jnp.int32)

        def z_body(t, _):
            cnt_v[pl.ds(t * LANES, LANES)] = zero
            return 0
        lax.fori_loop(0, NBKP // LANES, z_body, 0)

        iot = lax.iota(jnp.int32, LANES)

        def body(t, _):
            kvec = base_k + t * LANES + iot
            idxv = idx_v[pl.ds(t * LANES, LANES)]
            b = idxv >> 9
            cnt, last = plsc.scan_count(b)
            cur = plsc.load_gather(cnt_v, [b])
            rank = cur + cnt - 1
            plsc.store_scatter(cnt_v, [b], cur + cnt, mask=last)
            pk_v[pl.ds(t * LANES, LANES)] = (b << 13) | rank
            pv_v[pl.ds(t * LANES, LANES)] = ((idxv & (BSZ - 1)) << 18) | kvec
            return 0
        lax.fori_loop(0, iters, body, 0)

        ds_list = [
            pltpu.async_copy(pk_v, pk_hbm.at[pl.ds(base_k, per_w)], sem),
            pltpu.async_copy(pv_v, pv_hbm.at[pl.ds(base_k, per_w)], sem),
            pltpu.async_copy(
                cnt_v, counts_hbm.at[pl.ds(w * NBKP, NBKP)], sem),
        ]
        for d in ds_list:
            d.wait()

    return k1a(idx_flat)


def _place(counts, pk, pv, n):
    per_w = n // NW
    nbatch = per_w // 128
    mesh = plsc.VectorSubcoreMesh(core_axis_name="c", subcore_axis_name="s")

    @functools.partial(
        pl.kernel,
        mesh=mesh,
        out_type=(jax.ShapeDtypeStruct((n + 1024,), jnp.int32),
                  jax.ShapeDtypeStruct((NBKP + LANES,), jnp.int32)),
        scratch_types=[
            pltpu.VMEM((NW * NBKP,), jnp.int32),
            pltpu.VMEM((NBKP + LANES,), jnp.int32),
            pltpu.VMEM((NBKP,), jnp.int32),
            pltpu.VMEM((per_w,), jnp.int32),
            pltpu.VMEM((per_w,), jnp.int32),
            pltpu.VMEM((nbatch, 128), jnp.int32),
            pltpu.SemaphoreType.DMA,
        ],
        compiler_params=pltpu.CompilerParams(needs_layout_passes=False),
    )
    def k1b(counts_hbm, pk_hbm, pv_hbm, cells_hbm, pfx_hbm,
            cnts_v, tot_v, acc_v, pk_v, pv_v, abuf, sem):
        w = lax.axis_index("c") * 16 + lax.axis_index("s")
        base_k = w * per_w
        pltpu.sync_copy(counts_hbm, cnts_v)
        pltpu.sync_copy(pk_hbm.at[pl.ds(base_k, per_w)], pk_v)
        pltpu.sync_copy(pv_hbm.at[pl.ds(base_k, per_w)], pv_v)

        def sum_body(t, _):
            off = t * LANES
            tot = jnp.zeros((LANES,), jnp.int32)
            acc = jnp.zeros((LANES,), jnp.int32)
            for wp in range(NW):
                x = cnts_v[pl.ds(wp * NBKP + off, LANES)]
                tot = tot + x
                m = jnp.full((LANES,), (wp < w).astype(jnp.int32), jnp.int32)
                acc = acc + x * m
            tot_v[pl.ds(off, LANES)] = tot
            acc_v[pl.ds(off, LANES)] = acc
            return 0
        lax.fori_loop(0, NBKP // LANES, sum_body, 0)

        def scan_body(t, carry):
            x = tot_v[pl.ds(t * LANES, LANES)]
            inc = plsc.cumsum(x)
            tot_v[pl.ds(t * LANES, LANES)] = carry + inc - x
            return carry + jnp.full((LANES,), inc[15], jnp.int32)
        lax.fori_loop(0, NBKP // LANES, scan_body,
                      jnp.zeros((LANES,), jnp.int32))
        tot_v[pl.ds(NBKP, LANES)] = jnp.full((LANES,), n, jnp.int32)

        def body(t, _):
            pkv = pk_v[pl.ds(t * LANES, LANES)]
            b = pkv >> 13
            rank = pkv & 8191
            base = plsc.load_gather(tot_v, [b]) + plsc.load_gather(acc_v, [b])
            abuf[t >> 3, pl.ds((t & 7) * LANES, LANES)] = base + rank
            return 0
        lax.fori_loop(0, per_w // LANES, body, 0)

        seg = (NBKP + LANES) // NW
        ds_list = [pltpu.async_copy(
            tot_v.at[pl.ds(w * 64, 64)],
            pfx_hbm.at[pl.ds(w * 64, 64)], sem)]

        @pl.when(w == 0)
        def _():
            pltpu.sync_copy(tot_v.at[pl.ds(NBKP, LANES)],
                            pfx_hbm.at[pl.ds(NBKP, LANES)])
        for m in range(nbatch):
            ds_list.append(pltpu.async_copy(
                pv_v.at[pl.ds(m * 128, 128)],
                cells_hbm.at[abuf.at[m]], sem))
        for d in ds_list:
            d.wait()

    return k1b(counts, pk, pv)


def _stream_emit(tt, cells, pfx, pos_p, tail_p, n):
    nrows_out = n + NW
    CHW = 512
    mesh = plsc.VectorSubcoreMesh(core_axis_name="c", subcore_axis_name="s")

    @functools.partial(
        pl.kernel,
        mesh=mesh,
        out_type=jax.ShapeDtypeStruct((nrows_out, 128), jnp.float32),
        scratch_types=[
            pltpu.VMEM((2, 8, 8, 512), jnp.float32),
            pltpu.VMEM((96,), jnp.int32),
            pltpu.VMEM((CHW + 16,), jnp.int32),
            pltpu.VMEM((200, 128), jnp.float32),
            pltpu.VMEM((128, 128), jnp.float32),
            pltpu.VMEM((1, 128), jnp.int32),
            pltpu.SemaphoreType.DMA,
            pltpu.SemaphoreType.DMA,
        ],
        compiler_params=pltpu.CompilerParams(needs_layout_passes=False),
    )
    def k2(tt_hbm, cells_hbm, pfx_hbm, pos_hbm, tail_hbm, out_hbm,
           slab, pfx_v, cbuf, pos_v, stg, kl, sem, sem2):
        w = lax.axis_index("c") * 16 + lax.axis_index("s")
        b0 = (w * 977 + 15) >> 4
        b1 = ((w + 1) * 977 + 15) >> 4
        trash_row = n + w

        pltpu.sync_copy(pos_hbm, pos_v)
        a0 = pl.multiple_of(b0 & ~7, 8)
        pltpu.sync_copy(pfx_hbm.at[pl.ds(a0, 96)], pfx_v)

        iot = lax.iota(jnp.int32, LANES)
        fpat = [(j16 * LANES + iot) >> 3 for j16 in range(4)]
        rpat = [(j16 * LANES + iot) & 7 for j16 in range(4)]

        def fire_slab(b, half):
            @pl.when(b < NBK - 1)
            def _():
                for fb in range(8):
                    pltpu.async_copy(
                        tt_hbm.at[pl.ds(fb * 8, 8), pl.ds(b * BSZ, BSZ)],
                        slab.at[half, fb], sem)

            @pl.when(b == NBK - 1)
            def _():
                for fb in range(8):
                    pltpu.async_copy(
                        tail_hbm.at[pl.ds(fb * 8, 8)],
                        slab.at[half, fb, :, pl.ds(0, 128)], sem)

        def wait_slab(b, half):
            @pl.when(b < NBK - 1)
            def _():
                for fb in range(8):
                    pltpu.make_async_copy(
                        tt_hbm.at[pl.ds(fb * 8, 8), pl.ds(b * BSZ, BSZ)],
                        slab.at[half, fb], sem).wait()

            @pl.when(b == NBK - 1)
            def _():
                for fb in range(8):
                    pltpu.make_async_copy(
                        tail_hbm.at[pl.ds(fb * 8, 8)],
                        slab.at[half, fb, :, pl.ds(0, 128)], sem).wait()

        fire_slab(b0, 0)

        def bucket_loop(b, rs):
            half = (b - b0) & 1
            se = plsc.load_gather(pfx_v, [iot + (b - a0)])
            start = se[0]
            end = se[1]
            size = end - start
            al = pl.multiple_of(start & ~7, 8)

            wait_slab(b, half)

            @pl.when(b + 1 < b1)
            def _():
                fire_slab(b + 1, 1 - half)

            span = size + (start - al)
            nch = (span + CHW - 1) // CHW

            def chunk_loop(c, rs):
                cbase = pl.multiple_of(al + c * CHW, 8)
                pltpu.sync_copy(cells_hbm.at[pl.ds(cbase, CHW + 16)], cbuf)

                def emit_g(g, rs):
                    gpos = cbase + g * LANES + iot
                    valid = (gpos >= start) & (gpos < end)
                    vals = cbuf[pl.ds(g * LANES, LANES)]
                    vals = jnp.where(valid, vals, trash_row)
                    kkv = vals & 262143
                    ilv = vals >> 18
                    sv = kkv - 200 * ((kkv * 5243) >> 20)
                    sv = jnp.where(sv < 0, sv + 200, sv)
                    hbc = jnp.full((LANES,), half, jnp.int32)
                    for e in range(LANES):
                        s = sv[e]
                        ilb = jnp.full((LANES,), ilv[e], jnp.int32)
                        for j16 in range(4):
                            gv = plsc.load_gather(
                                slab, [hbc, fpat[j16], rpat[j16], ilb])
                            pv = pos_v[s, pl.ds(j16 * LANES, LANES)]
                            stg[rs + e, pl.ds(j16 * LANES, LANES)] = gv + pv
                    kl[0, pl.ds(pl.multiple_of(rs, LANES), LANES)] = kkv
                    rs = rs + LANES

                    @pl.when(rs == 128)
                    def _():
                        pltpu.async_copy(
                            stg, out_hbm.at[kl.at[0]], sem2).wait()
                    return jnp.where(rs == 128, 0, rs)

                ng = jnp.where(
                    c + 1 < nch, CHW // LANES,
                    (span - c * CHW + LANES - 1) // LANES)
                return lax.fori_loop(0, ng, emit_g, rs)

            return lax.fori_loop(0, nch, chunk_loop, rs)

        rs = lax.fori_loop(b0, b1, bucket_loop, jnp.int32(0))

        for h in range(8):
            lanes = h * LANES + iot
            curk = kl[0, pl.ds(h * LANES, LANES)]
            kl[0, pl.ds(h * LANES, LANES)] = \
                jnp.where(lanes < rs, curk, trash_row)
        pltpu.async_copy(stg, out_hbm.at[kl.at[0]], sem2).wait()

    return k2(tt, cells, pfx, pos_p, tail_p)


@jax.jit
def _run(x, token_table, pos_p, tail_p):
    B, S = x.shape
    n = B * S
    idx_flat = x.reshape(-1)
    counts, pk, pv = _count(idx_flat, n)
    cells, pfx = _place(counts, pk, pv, n)
    out128 = _stream_emit(token_table.T, cells, pfx, pos_p, tail_p, n)
    return out128[:n, :D].reshape(B, S, D)


def kernel(x, token_table):
    pos_enc = _pos_encoding_np(x.shape[1], D)
    pos_p = jnp.asarray(np.pad(pos_enc, ((0, 0), (0, 64))))
    tail_p = jnp.concatenate(
        [token_table[V - 64:].T,
         jnp.zeros((D, 64), jnp.float32)], axis=1)
    return _run(x, token_table, pos_p, tail_p)


# speedup vs baseline: 1.1250x; 1.1250x over previous
import functools

import jax
import jax.numpy as jnp
import numpy as np
from jax import lax
from jax.experimental import pallas as pl
from jax.experimental.pallas import tpu as pltpu
from jax.experimental.pallas import tpu_sc as plsc


def _pos_encoding_np(seq_len: int, d_model: int) -> np.ndarray:
    pos = np.arange(seq_len)[:, np.newaxis]
    i = np.arange(d_model)[np.newaxis, :]
    angle_rates = 1 / np.power(10000, 2 * (i // 2) / np.float32(d_model))
    angle_rads = pos * angle_rates
    angle_rads[:, 0::2] = np.sin(angle_rads[:, 0::2])
    angle_rads[:, 1::2] = np.cos(angle_rads[:, 1::2])
    return angle_rads.astype(np.float32)


@jax.jit
def _embed(x, token_table, pos_tiled):
    B, S = x.shape
    V, D = token_table.shape

    info = plsc.get_sparse_core_info()
    NC, NS = info.num_cores, info.num_subcores
    NW = NC * NS

    rows_total = B * S
    rows_per_w = rows_total // NW
    assert rows_per_w * NW == rows_total
    G = S // 2
    SEQ_PER_CHUNK = 8
    CH = SEQ_PER_CHUNK * S
    GPC = CH // G
    n_chunks = rows_per_w // CH
    assert n_chunks * CH == rows_per_w
    assert pos_tiled.shape == (CH, D)

    idx = x.reshape(NW, rows_per_w // G, G)

    mesh = plsc.VectorSubcoreMesh(core_axis_name="c", subcore_axis_name="s")

    @functools.partial(
        pl.kernel,
        mesh=mesh,
        out_type=jax.ShapeDtypeStruct((rows_total, D), jnp.float32),
        scratch_types=[
            pltpu.VMEM((rows_per_w // G, G), jnp.int32),
            pltpu.VMEM_SHARED((CH, D), jnp.float32),
            pltpu.VMEM((CH, D), jnp.float32),
            pltpu.SemaphoreType.DMA,
        ],
        compiler_params=pltpu.CompilerParams(use_tc_tiling_on_sc=False),
    )
    def k(table_hbm, idx_hbm, pos_hbm, out_hbm, idx_v, pos_sh, buf, sem):
        sid = lax.axis_index("s")
        wid = lax.axis_index("c") * NS + sid
        base = wid * rows_per_w

        @pl.when(sid == 0)
        def _():
            pltpu.sync_copy(pos_hbm, pos_sh)
        pltpu.sync_copy(idx_hbm.at[wid], idx_v)
        plsc.subcore_barrier()

        def chunk(c, _):
            pltpu.sync_copy(pos_sh, buf)
            ds = []
            for j in range(GPC):
                ds.append(pltpu.async_copy(
                    table_hbm.at[idx_v.at[c * GPC + j]],
                    buf.at[pl.ds(j * G, G)], sem, add=True))
            for d in ds:
                d.wait()
            pltpu.sync_copy(buf, out_hbm.at[pl.ds(base + c * CH, CH)])
            return 0

        lax.fori_loop(0, n_chunks, chunk, 0)

    return k(token_table, idx, pos_tiled)


def kernel(x, token_table):
    B, S = x.shape
    D = token_table.shape[1]
    pos_enc = _pos_encoding_np(S, D)
    pos_tiled = jnp.asarray(np.tile(pos_enc, (8, 1)))
    import os
    if not os.path.exists("/tmp/scband_problems/scband-token-and-position-embedding-79035988181043/hlo_dump.txt"):
        try:
            txt = _embed.lower(x, token_table, pos_tiled).compile().as_text()
            with open("/tmp/scband_problems/scband-token-and-position-embedding-79035988181043/hlo_dump.txt", "w") as f:
                f.write(txt)
        except Exception as e:
            with open("/tmp/scband_problems/scband-token-and-position-embedding-79035988181043/hlo_dump.txt", "w") as f:
                f.write(repr(e))
    out = _embed(x, token_table, pos_tiled)
    return out.reshape(B, S, D)


# speedup vs baseline: 1.1257x; 1.0006x over previous
import functools

import jax
import jax.numpy as jnp
import numpy as np
from jax import lax
from jax.experimental import pallas as pl
from jax.experimental.pallas import tpu as pltpu
from jax.experimental.pallas import tpu_sc as plsc


def _pos_encoding_np(seq_len: int, d_model: int) -> np.ndarray:
    pos = np.arange(seq_len)[:, np.newaxis]
    i = np.arange(d_model)[np.newaxis, :]
    angle_rates = 1 / np.power(10000, 2 * (i // 2) / np.float32(d_model))
    angle_rads = pos * angle_rates
    angle_rads[:, 0::2] = np.sin(angle_rads[:, 0::2])
    angle_rads[:, 1::2] = np.cos(angle_rads[:, 1::2])
    return angle_rads.astype(np.float32)


@jax.jit
def _embed(x, token_table, pos_tiled):
    B, S = x.shape
    V, D = token_table.shape

    info = plsc.get_sparse_core_info()
    NC, NS = info.num_cores, info.num_subcores
    NW = NC * NS

    rows_total = B * S
    rows_per_w = rows_total // NW
    assert rows_per_w * NW == rows_total
    G = S // 2
    SEQ_PER_CHUNK = 8
    CH = SEQ_PER_CHUNK * S
    GPC = CH // G
    n_chunks = rows_per_w // CH
    assert n_chunks * CH == rows_per_w
    assert pos_tiled.shape == (CH, D)

    idx = x.reshape(NW, rows_per_w // G, G)

    mesh = plsc.VectorSubcoreMesh(core_axis_name="c", subcore_axis_name="s")

    @functools.partial(
        pl.kernel,
        mesh=mesh,
        out_type=jax.ShapeDtypeStruct((rows_total, D), jnp.float32),
        scratch_types=[
            pltpu.VMEM((rows_per_w // G, G), jnp.int32),
            pltpu.VMEM_SHARED((CH, D), jnp.float32),
            pltpu.VMEM((CH, D), jnp.float32),
            pltpu.SemaphoreType.DMA,
        ],
        compiler_params=pltpu.CompilerParams(use_tc_tiling_on_sc=False),
    )
    def k(table_hbm, idx_hbm, pos_hbm, out_hbm, idx_v, pos_sh, buf, sem):
        sid = lax.axis_index("s")
        wid = lax.axis_index("c") * NS + sid
        base = wid * rows_per_w

        @pl.when(sid == 0)
        def _():
            pltpu.sync_copy(pos_hbm, pos_sh)
        pltpu.sync_copy(idx_hbm.at[wid], idx_v)
        plsc.subcore_barrier()

        def chunk(c, _):
            pltpu.sync_copy(pos_sh, buf)
            ds = []
            for j in range(GPC):
                ds.append(pltpu.async_copy(
                    table_hbm.at[idx_v.at[c * GPC + j]],
                    buf.at[pl.ds(j * G, G)], sem, add=True))
            for d in ds:
                d.wait()
            pltpu.sync_copy(buf, out_hbm.at[pl.ds(base + c * CH, CH)])
            return 0

        lax.fori_loop(0, n_chunks, chunk, 0)

    return k(token_table, idx, pos_tiled)


def kernel(x, token_table):
    B, S = x.shape
    D = token_table.shape[1]
    pos_enc = _pos_encoding_np(S, D)
    pos_tiled = jnp.asarray(np.tile(pos_enc, (8, 1)))
    out = _embed(x, token_table, pos_tiled)
    return out.reshape(B, S, D)
